# Initial kernel scaffold; baseline (speedup 1.0000x reference)
import jax
import jax.numpy as jnp
from jax.experimental import pallas as pl


def kernel(point_cls_preds, point_cls_labels):
    raise NotImplementedError("write your pallas kernel here")


# speedup vs baseline: 2.6922x; 2.6922x over previous
import functools

import jax
import jax.numpy as jnp
from jax import lax
from jax.experimental import pallas as pl
from jax.experimental.pallas import tpu as pltpu
from jax.experimental.pallas import tpu_sc as plsc

_NCLS = 3
_NC = 2
_NS = 16
_NW = _NC * _NS
_L = 16
_CHUNK = 4096


def _focal_partial_body(n_points, preds_hbm, labels_hbm, loss_out, pos_out,
                        preds_v, labels_v, stage_v):
    wid = lax.axis_index("s") * _NC + lax.axis_index("c")
    ppt = n_points // _NW
    nchunks = ppt // _CHUNK

    lanes = lax.iota(jnp.int32, _L)
    pidx_k = []
    cid_k = []
    for k in range(_NCLS):
        flat = lanes + (k * _L)
        q = lax.shift_right_logical(flat * 21846, 16)
        pidx_k.append(q)
        cid_k.append(flat - q * _NCLS + 1)

    def chunk_body(ci, acc):
        accl, accp = acc
        pbase = wid * ppt + ci * _CHUNK
        pltpu.sync_copy(preds_hbm.at[pl.ds(pbase * _NCLS, _CHUNK * _NCLS)],
                        preds_v)
        pltpu.sync_copy(labels_hbm.at[pl.ds(pbase, _CHUNK)], labels_v)

        def group_body(g, acc2):
            accl, accp = acc2
            lbl = labels_v[pl.ds(g * _L, _L)]
            accp = accp + jnp.where(lbl > 0, 1.0, 0.0)
            base = g * (_L * _NCLS)
            for k in range(_NCLS):
                p = preds_v[pl.ds(base + k * _L, _L)]
                lblk = lax.gather(
                    lbl, pidx_k[k][:, None],
                    lax.GatherDimensionNumbers(
                        offset_dims=(), collapsed_slice_dims=(0,),
                        start_index_map=(0,)),
                    slice_sizes=(1,),
                    mode=lax.GatherScatterMode.PROMISE_IN_BOUNDS)
                t = lblk == cid_k[k]
                z = jnp.where(t, -p, p)
                e = jnp.exp(-jnp.abs(z))
                inv = 1.0 / (1.0 + e)
                sig = jnp.where(z >= 0, inv, e * inv)
                y = e / (2.0 + e)
                y2 = y * y
                lg = y * (2.0 + y2 * (2.0 / 3.0 + y2 * (
                    2.0 / 5.0 + y2 * (2.0 / 7.0 + y2 * (2.0 / 9.0)))))
                sp = jnp.maximum(z, 0.0) + lg
                aw = jnp.where(t, 0.25, 0.75)
                accl = accl + aw * (sig * sig) * sp
            return accl, accp

        return lax.fori_loop(0, _CHUNK // _L, group_body, (accl, accp))

    zeros = jnp.zeros((_L,), jnp.float32)
    accl, accp = lax.fori_loop(0, nchunks, chunk_body, (zeros, zeros))
    stage_v[pl.ds(0, _L)] = accl
    stage_v[pl.ds(_L, _L)] = accp
    pltpu.sync_copy(stage_v.at[pl.ds(0, _L)], loss_out.at[pl.ds(wid * _L, _L)])
    pltpu.sync_copy(stage_v.at[pl.ds(_L, _L)], pos_out.at[pl.ds(wid * _L, _L)])


def _reduce_body(loss_hbm, pos_hbm, out_hbm, lv, pv, ov):
    wid = lax.axis_index("s") * _NC + lax.axis_index("c")

    @pl.when(wid == 0)
    def _():
        pltpu.sync_copy(loss_hbm, lv)
        pltpu.sync_copy(pos_hbm, pv)
        tl = jnp.zeros((_L,), jnp.float32)
        tp = jnp.zeros((_L,), jnp.float32)
        for i in range(_NW):
            tl = tl + lv[pl.ds(i * _L, _L)]
            tp = tp + pv[pl.ds(i * _L, _L)]

        lanes = lax.iota(jnp.int32, _L)

        def allsum(v):
            for sh in (8, 4, 2, 1):
                idx = lax.bitwise_xor(lanes, sh)
                v = v + lax.gather(
                    v, idx[:, None],
                    lax.GatherDimensionNumbers(
                        offset_dims=(), collapsed_slice_dims=(0,),
                        start_index_map=(0,)),
                    slice_sizes=(1,),
                    mode=lax.GatherScatterMode.PROMISE_IN_BOUNDS)
            return v

        r = allsum(tl) / jnp.maximum(allsum(tp), 1.0)
        ov[...] = r
        pltpu.sync_copy(ov, out_hbm)


@functools.lru_cache(maxsize=None)
def _build(n_points):
    mesh = plsc.VectorSubcoreMesh(core_axis_name="c", subcore_axis_name="s")
    partial_fn = functools.partial(
        pl.kernel,
        mesh=mesh,
        out_type=(
            jax.ShapeDtypeStruct((_NW * _L,), jnp.float32),
            jax.ShapeDtypeStruct((_NW * _L,), jnp.float32),
        ),
        scratch_types=[
            pltpu.VMEM((_CHUNK * _NCLS,), jnp.float32),
            pltpu.VMEM((_CHUNK,), jnp.int32),
            pltpu.VMEM((2 * _L,), jnp.float32),
        ],
    )(functools.partial(_focal_partial_body, n_points))
    reduce_fn = functools.partial(
        pl.kernel,
        mesh=mesh,
        out_type=jax.ShapeDtypeStruct((_L,), jnp.float32),
        scratch_types=[
            pltpu.VMEM((_NW * _L,), jnp.float32),
            pltpu.VMEM((_NW * _L,), jnp.float32),
            pltpu.VMEM((_L,), jnp.float32),
        ],
    )(_reduce_body)
    return partial_fn, reduce_fn


@jax.jit
def kernel(point_cls_preds, point_cls_labels):
    n = point_cls_labels.shape[0]
    partial_fn, reduce_fn = _build(n)
    preds_flat = point_cls_preds.reshape(-1)
    loss_p, pos_p = partial_fn(preds_flat, point_cls_labels)
    out = reduce_fn(loss_p, pos_p)
    return out[0]


# speedup vs baseline: 32.3701x; 12.0237x over previous
import functools

import jax
import jax.numpy as jnp
from jax import lax
from jax.experimental import pallas as pl
from jax.experimental.pallas import tpu as pltpu
from jax.experimental.pallas import tpu_sc as plsc

_NCLS = 3
_NC = 2
_NS = 16
_NW = _NC * _NS
_L = 16
_CHUNK = 4096


def _focal_partial_body(n_points, preds_hbm, labels_hbm, loss_out, pos_out,
                        preds_v, labels_v, stage_v):
    wid = lax.axis_index("s") * _NC + lax.axis_index("c")
    ppt = n_points // _NW
    nchunks = ppt // _CHUNK

    def chunk_body(ci, acc):
        accl, accp = acc
        pbase = wid * ppt + ci * _CHUNK
        for c in range(_NCLS):
            pltpu.sync_copy(preds_hbm.at[pl.ds(c * n_points + pbase, _CHUNK)],
                            preds_v.at[pl.ds(c * _CHUNK, _CHUNK)])
        pltpu.sync_copy(labels_hbm.at[pl.ds(pbase, _CHUNK)], labels_v)

        def group_body(g, acc2):
            accl, accp = acc2
            off = g * _L
            lbl = labels_v[pl.ds(off, _L)]
            accp = accp + jnp.where(lbl > 0, 1.0, 0.0)
            for c in range(_NCLS):
                p = preds_v[pl.ds(c * _CHUNK + off, _L)]
                t = lbl == (c + 1)
                z = jnp.where(t, -p, p)
                e = jnp.exp(-jnp.abs(z))
                inv = 1.0 / (1.0 + e)
                sig = jnp.where(z >= 0, inv, e * inv)
                y = e / (2.0 + e)
                y2 = y * y
                lg = y * (2.0 + y2 * (2.0 / 3.0 + y2 * (
                    2.0 / 5.0 + y2 * (2.0 / 7.0 + y2 * (2.0 / 9.0)))))
                sp = jnp.maximum(z, 0.0) + lg
                aw = jnp.where(t, 0.25, 0.75)
                accl = accl + aw * (sig * sig) * sp
            return accl, accp

        return lax.fori_loop(0, _CHUNK // _L, group_body, (accl, accp))

    zeros = jnp.zeros((_L,), jnp.float32)
    accl, accp = lax.fori_loop(0, nchunks, chunk_body, (zeros, zeros))
    stage_v[pl.ds(0, _L)] = accl
    stage_v[pl.ds(_L, _L)] = accp
    pltpu.sync_copy(stage_v.at[pl.ds(0, _L)], loss_out.at[pl.ds(wid * _L, _L)])
    pltpu.sync_copy(stage_v.at[pl.ds(_L, _L)], pos_out.at[pl.ds(wid * _L, _L)])


def _reduce_body(loss_hbm, pos_hbm, out_hbm, lv, pv, ov):
    wid = lax.axis_index("s") * _NC + lax.axis_index("c")

    @pl.when(wid == 0)
    def _():
        pltpu.sync_copy(loss_hbm, lv)
        pltpu.sync_copy(pos_hbm, pv)
        tl = jnp.zeros((_L,), jnp.float32)
        tp = jnp.zeros((_L,), jnp.float32)
        for i in range(_NW):
            tl = tl + lv[pl.ds(i * _L, _L)]
            tp = tp + pv[pl.ds(i * _L, _L)]

        lanes = lax.iota(jnp.int32, _L)

        def allsum(v):
            for sh in (8, 4, 2, 1):
                idx = lax.bitwise_xor(lanes, sh)
                v = v + lax.gather(
                    v, idx[:, None],
                    lax.GatherDimensionNumbers(
                        offset_dims=(), collapsed_slice_dims=(0,),
                        start_index_map=(0,)),
                    slice_sizes=(1,),
                    mode=lax.GatherScatterMode.PROMISE_IN_BOUNDS)
            return v

        r = allsum(tl) / jnp.maximum(allsum(tp), 1.0)
        ov[...] = r
        pltpu.sync_copy(ov, out_hbm)


@functools.lru_cache(maxsize=None)
def _build(n_points):
    mesh = plsc.VectorSubcoreMesh(core_axis_name="c", subcore_axis_name="s")
    partial_fn = functools.partial(
        pl.kernel,
        mesh=mesh,
        out_type=(
            jax.ShapeDtypeStruct((_NW * _L,), jnp.float32),
            jax.ShapeDtypeStruct((_NW * _L,), jnp.float32),
        ),
        scratch_types=[
            pltpu.VMEM((_CHUNK * _NCLS,), jnp.float32),
            pltpu.VMEM((_CHUNK,), jnp.int32),
            pltpu.VMEM((2 * _L,), jnp.float32),
        ],
    )(functools.partial(_focal_partial_body, n_points))
    reduce_fn = functools.partial(
        pl.kernel,
        mesh=mesh,
        out_type=jax.ShapeDtypeStruct((_L,), jnp.float32),
        scratch_types=[
            pltpu.VMEM((_NW * _L,), jnp.float32),
            pltpu.VMEM((_NW * _L,), jnp.float32),
            pltpu.VMEM((_L,), jnp.float32),
        ],
    )(_reduce_body)
    return partial_fn, reduce_fn


@jax.jit
def kernel(point_cls_preds, point_cls_labels):
    n = point_cls_labels.shape[0]
    partial_fn, reduce_fn = _build(n)
    preds_t = point_cls_preds.T.reshape(-1)
    loss_p, pos_p = partial_fn(preds_t, point_cls_labels)
    out = reduce_fn(loss_p, pos_p)
    return out[0]


# speedup vs baseline: 41.4482x; 1.2804x over previous
import functools

import jax
import jax.numpy as jnp
from jax import lax
from jax.experimental import pallas as pl
from jax.experimental.pallas import tpu as pltpu
from jax.experimental.pallas import tpu_sc as plsc

_NCLS = 3
_NC = 2
_NS = 16
_NW = _NC * _NS
_L = 16
_CHUNK = 8192
_UNROLL = 2


def _focal_group(p, lbl, c, accl):
    t = lbl == (c + 1)
    z = jnp.where(t, -p, p)
    e = jnp.exp(-jnp.abs(p))
    inv = 1.0 / (1.0 + e)
    sig = jnp.where(z >= 0, inv, e * inv)
    y = e / (2.0 + e)
    y2 = y * y
    lg = y * (2.0 + y2 * (2.0 / 3.0 + y2 * (
        2.0 / 5.0 + y2 * (2.0 / 7.0 + y2 * (2.0 / 9.0)))))
    sp = jnp.maximum(z, 0.0) + lg
    aw = jnp.where(t, 0.25, 0.75)
    return accl + aw * (sig * sig) * sp


def _focal_partial_body(n_points, preds_hbm, labels_hbm, part_out,
                        pv0, pv1, lv0, lv1, stage_v, sem0, sem1):
    wid = lax.axis_index("s") * _NC + lax.axis_index("c")
    ppt = n_points // _NW
    nchunks = ppt // _CHUNK
    bufs = ((pv0, lv0, sem0), (pv1, lv1, sem1))

    def start(ci, buf):
        pv, lv, sem = buf
        pbase = wid * ppt + ci * _CHUNK
        handles = []
        for c in range(_NCLS):
            handles.append(pltpu.async_copy(
                preds_hbm.at[pl.ds(c * n_points + pbase, _CHUNK)],
                pv.at[pl.ds(c * _CHUNK, _CHUNK)], sem))
        handles.append(pltpu.async_copy(
            labels_hbm.at[pl.ds(pbase, _CHUNK)], lv, sem))
        return handles

    pending = {0: start(0, bufs[0])}

    accl = jnp.zeros((_L,), jnp.float32)
    accp = jnp.zeros((_L,), jnp.float32)
    for ci in range(nchunks):
        b = ci % 2
        pv, lv, _ = bufs[b]
        for h in pending.pop(ci):
            h.wait()
        if ci + 1 < nchunks:
            pending[ci + 1] = start(ci + 1, bufs[1 - b])

        def group_body(g, acc, pv=pv, lv=lv):
            accl, accp = acc
            for u in range(_UNROLL):
                off = (g * _UNROLL + u) * _L
                lbl = lv[pl.ds(off, _L)]
                accp = accp + jnp.where(lbl > 0, 1.0, 0.0)
                for c in range(_NCLS):
                    p = pv[pl.ds(c * _CHUNK + off, _L)]
                    accl = _focal_group(p, lbl, c, accl)
            return accl, accp

        accl, accp = lax.fori_loop(0, _CHUNK // (_L * _UNROLL), group_body,
                                   (accl, accp))

    stage_v[pl.ds(0, _L)] = accl
    stage_v[pl.ds(_L, _L)] = accp
    pltpu.sync_copy(stage_v.at[pl.ds(0, _L)],
                    part_out.at[pl.ds(wid * _L, _L)])
    pltpu.sync_copy(stage_v.at[pl.ds(_L, _L)],
                    part_out.at[pl.ds(_NW * _L + wid * _L, _L)])


def _reduce_tc_body(part_ref, o_ref):
    part = part_ref[...]
    s = jnp.sum(part[:4, :])
    q = jnp.sum(part[4:, :])
    o_ref[...] = jnp.reshape(s / jnp.maximum(q, 1.0), (1, 1))


@functools.lru_cache(maxsize=None)
def _build(n_points):
    mesh = plsc.VectorSubcoreMesh(core_axis_name="c", subcore_axis_name="s")
    partial_fn = functools.partial(
        pl.kernel,
        mesh=mesh,
        out_type=jax.ShapeDtypeStruct((2 * _NW * _L,), jnp.float32),
        scratch_types=[
            pltpu.VMEM((_CHUNK * _NCLS,), jnp.float32),
            pltpu.VMEM((_CHUNK * _NCLS,), jnp.float32),
            pltpu.VMEM((_CHUNK,), jnp.int32),
            pltpu.VMEM((_CHUNK,), jnp.int32),
            pltpu.VMEM((2 * _L,), jnp.float32),
            pltpu.SemaphoreType.DMA,
            pltpu.SemaphoreType.DMA,
        ],
    )(functools.partial(_focal_partial_body, n_points))
    reduce_fn = pl.pallas_call(
        _reduce_tc_body,
        out_shape=jax.ShapeDtypeStruct((1, 1), jnp.float32),
    )
    return partial_fn, reduce_fn


@jax.jit
def kernel(point_cls_preds, point_cls_labels):
    n = point_cls_labels.shape[0]
    partial_fn, reduce_fn = _build(n)
    preds_t = point_cls_preds.T.reshape(-1)
    part = partial_fn(preds_t, point_cls_labels)
    out = reduce_fn(part.reshape(8, _NW * _L // 4))
    return out[0, 0]


# speedup vs baseline: 44.6120x; 1.0763x over previous
import functools

import jax
import jax.numpy as jnp
from jax import lax
from jax.experimental import pallas as pl
from jax.experimental.pallas import tpu as pltpu
from jax.experimental.pallas import tpu_sc as plsc

_NCLS = 3
_NC = 2
_NS = 16
_NW = _NC * _NS
_L = 16
_NSEG = 4
_CHUNK = 4096
_UNROLL = 2


def _focal_group(p, lbl, c, accl):
    t = lbl == (c + 1)
    z = jnp.where(t, -p, p)
    e = jnp.exp(-jnp.abs(p))
    inv = 1.0 / (1.0 + e)
    sig = jnp.where(z >= 0, inv, e * inv)
    y = e / (2.0 + e)
    y2 = y * y
    lg = y * (2.0 + y2 * (2.0 / 3.0 + y2 * (2.0 / 5.0)))
    sp = jnp.maximum(z, 0.0) + lg
    aw = jnp.where(t, 0.25, 0.75)
    return accl + aw * (sig * sig) * sp


def _focal_partial_body(n_seg, lbl_off, preds_hbm, labels_hbm, part_out,
                        pv0, pv1, lv0, lv1, stage_v, sem0, sem1):
    wid = lax.axis_index("s") * _NC + lax.axis_index("c")
    ppt = n_seg // _NW
    nchunks = ppt // _CHUNK
    bufs = ((pv0, lv0, sem0), (pv1, lv1, sem1))

    def start(ci, buf):
        pv, lv, sem = buf
        pbase = wid * ppt + ci * _CHUNK
        handles = []
        for c in range(_NCLS):
            handles.append(pltpu.async_copy(
                preds_hbm.at[pl.ds(c * n_seg + pbase, _CHUNK)],
                pv.at[pl.ds(c * _CHUNK, _CHUNK)], sem))
        handles.append(pltpu.async_copy(
            labels_hbm.at[pl.ds(lbl_off + pbase, _CHUNK)], lv, sem))
        return handles

    pending = {0: start(0, bufs[0])}

    accl = jnp.zeros((_L,), jnp.float32)
    accp = jnp.zeros((_L,), jnp.float32)
    for ci in range(nchunks):
        b = ci % 2
        pv, lv, _ = bufs[b]
        for h in pending.pop(ci):
            h.wait()
        if ci + 1 < nchunks:
            pending[ci + 1] = start(ci + 1, bufs[1 - b])

        def group_body(g, acc, pv=pv, lv=lv):
            accl, accp = acc
            for u in range(_UNROLL):
                off = (g * _UNROLL + u) * _L
                lbl = lv[pl.ds(off, _L)]
                accp = accp + jnp.where(lbl > 0, 1.0, 0.0)
                for c in range(_NCLS):
                    p = pv[pl.ds(c * _CHUNK + off, _L)]
                    accl = _focal_group(p, lbl, c, accl)
            return accl, accp

        accl, accp = lax.fori_loop(0, _CHUNK // (_L * _UNROLL), group_body,
                                   (accl, accp))

    stage_v[pl.ds(0, _L)] = accl
    stage_v[pl.ds(_L, _L)] = accp
    pltpu.sync_copy(stage_v.at[pl.ds(0, _L)],
                    part_out.at[pl.ds(wid * _L, _L)])
    pltpu.sync_copy(stage_v.at[pl.ds(_L, _L)],
                    part_out.at[pl.ds(_NW * _L + wid * _L, _L)])


def _reduce_tc_body(*refs):
    parts = refs[:-1]
    o_ref = refs[-1]
    s = jnp.zeros((), jnp.float32)
    q = jnp.zeros((), jnp.float32)
    for p_ref in parts:
        part = p_ref[...]
        s = s + jnp.sum(part[:4, :])
        q = q + jnp.sum(part[4:, :])
    o_ref[...] = jnp.reshape(s / jnp.maximum(q, 1.0), (1, 1))


@functools.lru_cache(maxsize=None)
def _build(n_points):
    n_seg = n_points // _NSEG
    mesh = plsc.VectorSubcoreMesh(core_axis_name="c", subcore_axis_name="s")
    seg_fns = []
    for s in range(_NSEG):
        seg_fns.append(functools.partial(
            pl.kernel,
            mesh=mesh,
            out_type=jax.ShapeDtypeStruct((2 * _NW * _L,), jnp.float32),
            scratch_types=[
                pltpu.VMEM((_CHUNK * _NCLS,), jnp.float32),
                pltpu.VMEM((_CHUNK * _NCLS,), jnp.float32),
                pltpu.VMEM((_CHUNK,), jnp.int32),
                pltpu.VMEM((_CHUNK,), jnp.int32),
                pltpu.VMEM((2 * _L,), jnp.float32),
                pltpu.SemaphoreType.DMA,
                pltpu.SemaphoreType.DMA,
            ],
        )(functools.partial(_focal_partial_body, n_seg, s * n_seg)))
    reduce_fn = pl.pallas_call(
        _reduce_tc_body,
        out_shape=jax.ShapeDtypeStruct((1, 1), jnp.float32),
    )
    return seg_fns, reduce_fn


@jax.jit
def kernel(point_cls_preds, point_cls_labels):
    n = point_cls_labels.shape[0]
    n_seg = n // _NSEG
    seg_fns, reduce_fn = _build(n)
    preds_t = point_cls_preds.T
    parts = []
    for s, fn in enumerate(seg_fns):
        seg = preds_t[:, s * n_seg:(s + 1) * n_seg].reshape(-1)
        parts.append(fn(seg, point_cls_labels))
    out = reduce_fn(*[p.reshape(8, _NW * _L // 4) for p in parts])
    return out[0, 0]
